# baseline (speedup 1.0000x reference)
import functools

import jax
import jax.numpy as jnp
from jax import lax
from jax.experimental import pallas as pl
from jax.experimental.pallas import tpu as pltpu
from jax.experimental.pallas import tpu_sc as plsc

NC = 2
NS = 16
NW = NC * NS
LANES = 16

N_NODES = 10000
TPR = 320
NPAD = NW * TPR
ROWS = TPR + 8
KB = 128
CHUNK = 6400


def _sc_params():
    return pltpu.CompilerParams(needs_layout_passes=False)


def _mesh():
    return plsc.VectorSubcoreMesh(
        core_axis_name="c", subcore_axis_name="s",
        num_cores=NC, num_subcores=NS)


def _make_partition(n_edges):
    cap = n_edges + 2 * KB
    assert cap % KB == 0 and n_edges % CHUNK == 0
    nchunks = n_edges // CHUNK
    nsub = CHUNK // KB

    @functools.partial(
        pl.kernel,
        out_type=[
            jax.ShapeDtypeStruct((NW * cap,), jnp.int32),
            jax.ShapeDtypeStruct((NW * cap,), jnp.int32),
            jax.ShapeDtypeStruct((NW * LANES,), jnp.int32),
            jax.ShapeDtypeStruct((NPAD * LANES,), jnp.float32),
        ],
        mesh=_mesh(),
        scratch_types=[
            pltpu.VMEM((CHUNK,), jnp.int32),
            pltpu.VMEM((CHUNK,), jnp.int32),
            pltpu.VMEM((2 * KB,), jnp.int32),
            pltpu.VMEM((2 * KB,), jnp.int32),
            pltpu.VMEM((ROWS * LANES,), jnp.float32),
        ],
        compiler_params=_sc_params(),
    )
    def part(src_hbm, dst_hbm, csrc_hbm, cloc_hbm, nblk_hbm, deg_hbm,
             sbuf, dbuf, ssrc, sloc, degacc):
        cid = lax.axis_index("c")
        sid = lax.axis_index("s")
        w = cid * NS + sid
        base = w * TPR
        out0 = w * cap
        iota = lax.iota(jnp.int32, LANES)
        ones = jnp.ones((LANES,), jnp.float32)
        zeros = jnp.zeros((LANES,), jnp.float32)

        for r in range(ROWS):
            degacc[pl.ds(r * LANES, LANES)] = zeros

        def flush(carry):
            cnt, blk = carry
            off = out0 + blk * KB
            pltpu.sync_copy(ssrc.at[pl.ds(0, KB)], csrc_hbm.at[pl.ds(off, KB)])
            pltpu.sync_copy(sloc.at[pl.ds(0, KB)], cloc_hbm.at[pl.ds(off, KB)])
            for g in range(KB // LANES):
                ssrc[pl.ds(g * LANES, LANES)] = ssrc[pl.ds(KB + g * LANES, LANES)]
                sloc[pl.ds(g * LANES, LANES)] = sloc[pl.ds(KB + g * LANES, LANES)]
            return cnt - KB, blk + 1

        def chunk_body(i, carry):
            pltpu.sync_copy(src_hbm.at[pl.ds(i * CHUNK, CHUNK)], sbuf)
            pltpu.sync_copy(dst_hbm.at[pl.ds(i * CHUNK, CHUNK)], dbuf)

            def sub_body(s, carry):
                cnt, blk = carry
                for g in range(KB // LANES):
                    o = s * KB + g * LANES
                    d16 = dbuf[pl.ds(o, LANES)]
                    s16 = sbuf[pl.ds(o, LANES)]
                    loc = d16 - base
                    m = plsc.bitcast(loc, jnp.uint32) < jnp.uint32(TPR)
                    pos = plsc.cumsum(jnp.where(m, 1, 0))
                    tgt = cnt + pos - 1
                    plsc.store_scatter(ssrc, [tgt], s16, mask=m)
                    plsc.store_scatter(sloc, [tgt], loc, mask=m)
                    locs = jnp.where(m, loc, TPR)
                    plsc.addupdate_scatter(degacc, [locs * LANES + iota], ones)
                    cnt = cnt + plsc.all_reduce_population_count(m)[0]
                return lax.cond(cnt >= KB, flush, lambda c: c, (cnt, blk))

            return lax.fori_loop(0, nsub, sub_body, carry)

        cnt, blk = lax.fori_loop(0, nchunks, chunk_body, (0, 0))

        for g in range(2 * KB // LANES):
            gi = iota + g * LANES
            pad = gi >= cnt
            s16 = ssrc[pl.ds(g * LANES, LANES)]
            l16 = sloc[pl.ds(g * LANES, LANES)]
            ssrc[pl.ds(g * LANES, LANES)] = jnp.where(pad, 0, s16)
            sloc[pl.ds(g * LANES, LANES)] = jnp.where(pad, TPR, l16)
        off = out0 + blk * KB
        pltpu.sync_copy(ssrc, csrc_hbm.at[pl.ds(off, 2 * KB)])
        pltpu.sync_copy(sloc, cloc_hbm.at[pl.ds(off, 2 * KB)])

        sbuf[pl.ds(0, LANES)] = jnp.zeros((LANES,), jnp.int32) + blk + 2
        pltpu.sync_copy(sbuf.at[pl.ds(0, LANES)],
                        nblk_hbm.at[pl.ds(w * LANES, LANES)])
        pltpu.sync_copy(degacc.at[pl.ds(0, TPR * LANES)],
                        deg_hbm.at[pl.ds(w * TPR * LANES, TPR * LANES)])

    return part, cap


def _make_agg(n_nodes, cap, d, eb):
    assert d % 128 == 0 and KB % eb == 0

    @functools.partial(
        pl.kernel,
        out_type=jax.ShapeDtypeStruct((NPAD, d), jnp.float32),
        mesh=_mesh(),
        scratch_types=[
            pltpu.VMEM((ROWS, d), jnp.float32),
            pltpu.VMEM((2, eb), jnp.int32),
            pltpu.VMEM((2, eb), jnp.int32),
            pltpu.VMEM((2, eb, d), jnp.float32),
            pltpu.VMEM((LANES,), jnp.int32),
            pltpu.SemaphoreType.DMA,
            pltpu.SemaphoreType.DMA,
        ],
        compiler_params=_sc_params(),
    )
    def agg(table_hbm, csrc_hbm, cloc_hbm, nblk_hbm, out_hbm,
            acc, sidx, lbuf, rbuf, nbuf, sem0, sem1):
        cid = lax.axis_index("c")
        sid = lax.axis_index("s")
        w = cid * NS + sid
        out0 = w * cap
        zeros = jnp.zeros((LANES,), jnp.float32)
        sems = (sem0, sem1)

        for r in range(ROWS):
            for ch in range(d // LANES):
                acc[r, pl.ds(ch * LANES, LANES)] = zeros

        pltpu.sync_copy(nblk_hbm.at[pl.ds(w * LANES, LANES)], nbuf)
        nb = nbuf[pl.ds(0, LANES)][0] * (KB // eb)

        def issue(b, k):
            off = out0 + b * eb
            pltpu.sync_copy(csrc_hbm.at[pl.ds(off, eb)], sidx.at[k])
            pltpu.sync_copy(cloc_hbm.at[pl.ds(off, eb)], lbuf.at[k])
            pltpu.async_copy(table_hbm.at[sidx.at[k]], rbuf.at[k], sems[k])

        def wait(k):
            pltpu.make_async_copy(table_hbm.at[sidx.at[k]], rbuf.at[k],
                                  sems[k]).wait()

        def process(k):
            iota = lax.iota(jnp.int32, LANES)
            cols = [ch * LANES + iota for ch in range(d // LANES)]

            def edge_body(j, c2):
                locv = plsc.load_gather(lbuf.at[k], [iota * 0 + j])
                vs = [rbuf[k, j, pl.ds(ch * LANES, LANES)]
                      for ch in range(d // LANES)]
                for ch in range(d // LANES):
                    plsc.addupdate_scatter(acc, [locv, cols[ch]], vs[ch])
                return c2

            lax.fori_loop(0, eb, edge_body, 0, unroll=4)

        issue(0, 0)

        def pair_body(p, carry):
            b0 = 2 * p
            wait(0)

            @pl.when(b0 + 1 < nb)
            def _():
                issue(b0 + 1, 1)

            process(0)

            @pl.when(b0 + 1 < nb)
            def _():
                wait(1)

                @pl.when(b0 + 2 < nb)
                def _():
                    issue(b0 + 2, 0)

                process(1)

            return carry

        lax.fori_loop(0, (nb + 1) // 2, pair_body, 0)

        pltpu.sync_copy(acc.at[pl.ds(0, TPR)],
                        out_hbm.at[pl.ds(w * TPR, TPR)])

    return agg


_BR = 1000


def _scale0(x, deg16):
    n, din = x.shape

    def body(x_ref, deg_ref, g_ref):
        deg = jnp.sum(deg_ref[...], axis=1, keepdims=True)
        dis = lax.rsqrt(deg + 1.0)
        g_ref[...] = x_ref[...] * dis

    return pl.pallas_call(
        body,
        grid=(n // _BR,),
        in_specs=[
            pl.BlockSpec((_BR, din), lambda i: (i, 0)),
            pl.BlockSpec((_BR, LANES), lambda i: (i, 0)),
        ],
        out_specs=pl.BlockSpec((_BR, din), lambda i: (i, 0)),
        out_shape=jax.ShapeDtypeStruct((n, din), jnp.float32),
    )(x, deg16)


def _fused_mm(a1, x, deg16, b1, w1, w2):
    n, din = x.shape
    dmid = w1.shape[1]
    dout = w2.shape[1]

    def body(a_ref, x_ref, deg_ref, b_ref, w1_ref, w2_ref, h2_ref, g2_ref):
        deg = jnp.sum(deg_ref[...], axis=1, keepdims=True)
        dis = lax.rsqrt(deg + 1.0)
        pre = dis * a_ref[...] + (dis * dis) * x_ref[...]
        o1 = jnp.maximum(
            jnp.dot(pre, w1_ref[...], preferred_element_type=jnp.float32)
            + b_ref[...], 0.0)
        h2 = jnp.dot(o1, w2_ref[...], preferred_element_type=jnp.float32)
        h2_ref[...] = h2
        g2_ref[...] = h2 * dis

    return pl.pallas_call(
        body,
        grid=(n // _BR,),
        in_specs=[
            pl.BlockSpec((_BR, din), lambda i: (i, 0)),
            pl.BlockSpec((_BR, din), lambda i: (i, 0)),
            pl.BlockSpec((_BR, LANES), lambda i: (i, 0)),
            pl.BlockSpec((1, dmid), lambda i: (0, 0)),
            pl.BlockSpec((din, dmid), lambda i: (0, 0)),
            pl.BlockSpec((dmid, dout), lambda i: (0, 0)),
        ],
        out_specs=[
            pl.BlockSpec((_BR, dout), lambda i: (i, 0)),
            pl.BlockSpec((_BR, dout), lambda i: (i, 0)),
        ],
        out_shape=[
            jax.ShapeDtypeStruct((n, dout), jnp.float32),
            jax.ShapeDtypeStruct((n, dout), jnp.float32),
        ],
    )(a1, x, deg16, b1, w1, w2)


def _combine_final(aggv, h, deg16, b):
    n, dout = h.shape

    def body(a_ref, h_ref, deg_ref, b_ref, o_ref):
        deg = jnp.sum(deg_ref[...], axis=1, keepdims=True)
        dis = lax.rsqrt(deg + 1.0)
        o_ref[...] = dis * a_ref[...] + (dis * dis) * h_ref[...] + b_ref[...]

    return pl.pallas_call(
        body,
        grid=(n // _BR,),
        in_specs=[
            pl.BlockSpec((_BR, dout), lambda i: (i, 0)),
            pl.BlockSpec((_BR, dout), lambda i: (i, 0)),
            pl.BlockSpec((_BR, LANES), lambda i: (i, 0)),
            pl.BlockSpec((1, dout), lambda i: (0, 0)),
        ],
        out_specs=pl.BlockSpec((_BR, dout), lambda i: (i, 0)),
        out_shape=jax.ShapeDtypeStruct((n, dout), jnp.float32),
    )(aggv, h, deg16, b)


def kernel(x, edge_index, W1, b1, W2, b2):
    n, _ = x.shape
    e = edge_index.shape[1]
    src = edge_index[0].astype(jnp.int32)
    dst = edge_index[1].astype(jnp.int32)

    part, cap = _make_partition(e)
    csrc, cloc, nblk, deg_flat = part(src, dst)
    deg16 = deg_flat.reshape(NPAD, LANES)[:n]

    agg = _make_agg(n, cap, x.shape[1], 128)

    g0 = _scale0(x, deg16)
    a1 = agg(g0, csrc, cloc, nblk)[:n]
    h2, g2 = _fused_mm(a1, x, deg16, b1.reshape(1, -1), W1, W2)
    a2 = agg(g2, csrc, cloc, nblk)[:n]
    return _combine_final(a2, h2, deg16, b2.reshape(1, -1))


# speedup vs baseline: 1.0538x; 1.0538x over previous
import functools

import jax
import jax.numpy as jnp
from jax import lax
from jax.experimental import pallas as pl
from jax.experimental.pallas import tpu as pltpu
from jax.experimental.pallas import tpu_sc as plsc

NC = 2
NS = 16
NW = NC * NS
LANES = 16

N_NODES = 10000
TPR = 320
NPAD = NW * TPR
ROWS = TPR + 8
KB = 128
CHUNK = 6400


def _sc_params():
    return pltpu.CompilerParams(needs_layout_passes=False)


def _mesh():
    return plsc.VectorSubcoreMesh(
        core_axis_name="c", subcore_axis_name="s",
        num_cores=NC, num_subcores=NS)


def _make_partition(n_edges):
    cap = n_edges + 2 * KB
    assert cap % KB == 0 and n_edges % CHUNK == 0
    nchunks = n_edges // CHUNK
    nsub = CHUNK // KB

    @functools.partial(
        pl.kernel,
        out_type=[
            jax.ShapeDtypeStruct((NW * cap,), jnp.int32),
            jax.ShapeDtypeStruct((NW * cap,), jnp.int32),
            jax.ShapeDtypeStruct((NW * LANES,), jnp.int32),
            jax.ShapeDtypeStruct((NPAD * LANES,), jnp.float32),
        ],
        mesh=_mesh(),
        scratch_types=[
            pltpu.VMEM((2, CHUNK), jnp.int32),
            pltpu.VMEM((2, CHUNK), jnp.int32),
            pltpu.VMEM((2 * KB,), jnp.int32),
            pltpu.VMEM((2 * KB,), jnp.int32),
            pltpu.VMEM((ROWS * LANES,), jnp.float32),
            pltpu.SemaphoreType.DMA,
            pltpu.SemaphoreType.DMA,
        ],
        compiler_params=_sc_params(),
    )
    def part(src_hbm, dst_hbm, csrc_hbm, cloc_hbm, nblk_hbm, deg_hbm,
             sbuf, dbuf, ssrc, sloc, degacc, csem0, csem1):
        cid = lax.axis_index("c")
        sid = lax.axis_index("s")
        w = cid * NS + sid
        base = w * TPR
        out0 = w * cap
        iota = lax.iota(jnp.int32, LANES)
        ones = jnp.ones((LANES,), jnp.float32)
        zeros = jnp.zeros((LANES,), jnp.float32)

        for r in range(ROWS):
            degacc[pl.ds(r * LANES, LANES)] = zeros

        def flush(carry):
            cnt, blk = carry
            off = out0 + blk * KB
            pltpu.sync_copy(ssrc.at[pl.ds(0, KB)], csrc_hbm.at[pl.ds(off, KB)])
            pltpu.sync_copy(sloc.at[pl.ds(0, KB)], cloc_hbm.at[pl.ds(off, KB)])
            for g in range(KB // LANES):
                ssrc[pl.ds(g * LANES, LANES)] = ssrc[pl.ds(KB + g * LANES, LANES)]
                sloc[pl.ds(g * LANES, LANES)] = sloc[pl.ds(KB + g * LANES, LANES)]
            return cnt - KB, blk + 1

        csems = (csem0, csem1)

        def issue_chunk(i, kb):
            pltpu.async_copy(src_hbm.at[pl.ds(i * CHUNK, CHUNK)],
                             sbuf.at[kb], csems[kb])
            pltpu.async_copy(dst_hbm.at[pl.ds(i * CHUNK, CHUNK)],
                             dbuf.at[kb], csems[kb])

        def wait_chunk(i, kb):
            pltpu.make_async_copy(src_hbm.at[pl.ds(i * CHUNK, CHUNK)],
                                  sbuf.at[kb], csems[kb]).wait()
            pltpu.make_async_copy(dst_hbm.at[pl.ds(i * CHUNK, CHUNK)],
                                  dbuf.at[kb], csems[kb]).wait()

        def process_chunk(kb, carry):
            def sub_body(s, carry):
                cnt, blk = carry
                for g in range(KB // LANES):
                    o = s * KB + g * LANES
                    d16 = dbuf[kb, pl.ds(o, LANES)]
                    s16 = sbuf[kb, pl.ds(o, LANES)]
                    loc = d16 - base
                    m = plsc.bitcast(loc, jnp.uint32) < jnp.uint32(TPR)
                    pos = plsc.cumsum(jnp.where(m, 1, 0))
                    tgt = cnt + pos - 1
                    plsc.store_scatter(ssrc, [tgt], s16, mask=m)
                    plsc.store_scatter(sloc, [tgt], loc, mask=m)
                    locs = jnp.where(m, loc, TPR)
                    plsc.addupdate_scatter(degacc, [locs * LANES + iota], ones)
                    cnt = cnt + plsc.all_reduce_population_count(m)[0]
                return lax.cond(cnt >= KB, flush, lambda c: c, (cnt, blk))

            return lax.fori_loop(0, nsub, sub_body, carry)

        assert nchunks % 2 == 0
        issue_chunk(0, 0)

        def chunk_pair(p, carry):
            i0 = 2 * p
            wait_chunk(i0, 0)

            @pl.when(i0 + 1 < nchunks)
            def _():
                issue_chunk(i0 + 1, 1)

            carry = process_chunk(0, carry)
            wait_chunk(i0 + 1, 1)

            @pl.when(i0 + 2 < nchunks)
            def _():
                issue_chunk(i0 + 2, 0)

            return process_chunk(1, carry)

        cnt, blk = lax.fori_loop(0, nchunks // 2, chunk_pair, (0, 0))

        for g in range(2 * KB // LANES):
            gi = iota + g * LANES
            pad = gi >= cnt
            s16 = ssrc[pl.ds(g * LANES, LANES)]
            l16 = sloc[pl.ds(g * LANES, LANES)]
            ssrc[pl.ds(g * LANES, LANES)] = jnp.where(pad, 0, s16)
            sloc[pl.ds(g * LANES, LANES)] = jnp.where(pad, TPR, l16)
        off = out0 + blk * KB
        pltpu.sync_copy(ssrc, csrc_hbm.at[pl.ds(off, 2 * KB)])
        pltpu.sync_copy(sloc, cloc_hbm.at[pl.ds(off, 2 * KB)])

        sbuf[0, pl.ds(0, LANES)] = jnp.zeros((LANES,), jnp.int32) + blk + 2
        pltpu.sync_copy(sbuf.at[0, pl.ds(0, LANES)],
                        nblk_hbm.at[pl.ds(w * LANES, LANES)])
        pltpu.sync_copy(degacc.at[pl.ds(0, TPR * LANES)],
                        deg_hbm.at[pl.ds(w * TPR * LANES, TPR * LANES)])

    return part, cap


def _make_agg(n_nodes, cap, d, eb):
    assert d % 128 == 0 and KB % eb == 0

    @functools.partial(
        pl.kernel,
        out_type=jax.ShapeDtypeStruct((NPAD, d), jnp.float32),
        mesh=_mesh(),
        scratch_types=[
            pltpu.VMEM((ROWS, d), jnp.float32),
            pltpu.VMEM((2, eb), jnp.int32),
            pltpu.VMEM((2, eb), jnp.int32),
            pltpu.VMEM((2, eb, d), jnp.float32),
            pltpu.VMEM((LANES,), jnp.int32),
            pltpu.SemaphoreType.DMA,
            pltpu.SemaphoreType.DMA,
        ],
        compiler_params=_sc_params(),
    )
    def agg(table_hbm, csrc_hbm, cloc_hbm, nblk_hbm, out_hbm,
            acc, sidx, lbuf, rbuf, nbuf, sem0, sem1):
        cid = lax.axis_index("c")
        sid = lax.axis_index("s")
        w = cid * NS + sid
        out0 = w * cap
        zeros = jnp.zeros((LANES,), jnp.float32)
        sems = (sem0, sem1)

        for r in range(ROWS):
            for ch in range(d // LANES):
                acc[r, pl.ds(ch * LANES, LANES)] = zeros

        pltpu.sync_copy(nblk_hbm.at[pl.ds(w * LANES, LANES)], nbuf)
        nb = nbuf[pl.ds(0, LANES)][0] * (KB // eb)

        def issue(b, k):
            off = out0 + b * eb
            pltpu.sync_copy(csrc_hbm.at[pl.ds(off, eb)], sidx.at[k])
            pltpu.sync_copy(cloc_hbm.at[pl.ds(off, eb)], lbuf.at[k])
            pltpu.async_copy(table_hbm.at[sidx.at[k]], rbuf.at[k], sems[k])

        def wait(k):
            pltpu.make_async_copy(table_hbm.at[sidx.at[k]], rbuf.at[k],
                                  sems[k]).wait()

        def process(k):
            iota = lax.iota(jnp.int32, LANES)
            cols = [ch * LANES + iota for ch in range(d // LANES)]

            def edge_body(j, c2):
                locv = plsc.load_gather(lbuf.at[k], [iota * 0 + j])
                vs = [rbuf[k, j, pl.ds(ch * LANES, LANES)]
                      for ch in range(d // LANES)]
                for ch in range(d // LANES):
                    plsc.addupdate_scatter(acc, [locv, cols[ch]], vs[ch])
                return c2

            lax.fori_loop(0, eb, edge_body, 0)

        issue(0, 0)

        def pair_body(p, carry):
            b0 = 2 * p
            wait(0)

            @pl.when(b0 + 1 < nb)
            def _():
                issue(b0 + 1, 1)

            process(0)

            @pl.when(b0 + 1 < nb)
            def _():
                wait(1)

                @pl.when(b0 + 2 < nb)
                def _():
                    issue(b0 + 2, 0)

                process(1)

            return carry

        lax.fori_loop(0, (nb + 1) // 2, pair_body, 0)

        pltpu.sync_copy(acc.at[pl.ds(0, TPR)],
                        out_hbm.at[pl.ds(w * TPR, TPR)])

    return agg


_BR = 1000


def _scale0(x, deg16):
    n, din = x.shape

    def body(x_ref, deg_ref, g_ref):
        deg = jnp.sum(deg_ref[...], axis=1, keepdims=True)
        dis = lax.rsqrt(deg + 1.0)
        g_ref[...] = x_ref[...] * dis

    return pl.pallas_call(
        body,
        grid=(n // _BR,),
        in_specs=[
            pl.BlockSpec((_BR, din), lambda i: (i, 0)),
            pl.BlockSpec((_BR, LANES), lambda i: (i, 0)),
        ],
        out_specs=pl.BlockSpec((_BR, din), lambda i: (i, 0)),
        out_shape=jax.ShapeDtypeStruct((n, din), jnp.float32),
    )(x, deg16)


def _fused_mm(a1, x, deg16, b1, w1, w2):
    n, din = x.shape
    dmid = w1.shape[1]
    dout = w2.shape[1]

    def body(a_ref, x_ref, deg_ref, b_ref, w1_ref, w2_ref, h2_ref, g2_ref):
        deg = jnp.sum(deg_ref[...], axis=1, keepdims=True)
        dis = lax.rsqrt(deg + 1.0)
        pre = dis * a_ref[...] + (dis * dis) * x_ref[...]
        o1 = jnp.maximum(
            jnp.dot(pre, w1_ref[...], preferred_element_type=jnp.float32)
            + b_ref[...], 0.0)
        h2 = jnp.dot(o1, w2_ref[...], preferred_element_type=jnp.float32)
        h2_ref[...] = h2
        g2_ref[...] = h2 * dis

    return pl.pallas_call(
        body,
        grid=(n // _BR,),
        in_specs=[
            pl.BlockSpec((_BR, din), lambda i: (i, 0)),
            pl.BlockSpec((_BR, din), lambda i: (i, 0)),
            pl.BlockSpec((_BR, LANES), lambda i: (i, 0)),
            pl.BlockSpec((1, dmid), lambda i: (0, 0)),
            pl.BlockSpec((din, dmid), lambda i: (0, 0)),
            pl.BlockSpec((dmid, dout), lambda i: (0, 0)),
        ],
        out_specs=[
            pl.BlockSpec((_BR, dout), lambda i: (i, 0)),
            pl.BlockSpec((_BR, dout), lambda i: (i, 0)),
        ],
        out_shape=[
            jax.ShapeDtypeStruct((n, dout), jnp.float32),
            jax.ShapeDtypeStruct((n, dout), jnp.float32),
        ],
    )(a1, x, deg16, b1, w1, w2)


def _combine_final(aggv, h, deg16, b):
    n, dout = h.shape

    def body(a_ref, h_ref, deg_ref, b_ref, o_ref):
        deg = jnp.sum(deg_ref[...], axis=1, keepdims=True)
        dis = lax.rsqrt(deg + 1.0)
        o_ref[...] = dis * a_ref[...] + (dis * dis) * h_ref[...] + b_ref[...]

    return pl.pallas_call(
        body,
        grid=(n // _BR,),
        in_specs=[
            pl.BlockSpec((_BR, dout), lambda i: (i, 0)),
            pl.BlockSpec((_BR, dout), lambda i: (i, 0)),
            pl.BlockSpec((_BR, LANES), lambda i: (i, 0)),
            pl.BlockSpec((1, dout), lambda i: (0, 0)),
        ],
        out_specs=pl.BlockSpec((_BR, dout), lambda i: (i, 0)),
        out_shape=jax.ShapeDtypeStruct((n, dout), jnp.float32),
    )(aggv, h, deg16, b)


def kernel(x, edge_index, W1, b1, W2, b2):
    n, _ = x.shape
    e = edge_index.shape[1]
    src = edge_index[0].astype(jnp.int32)
    dst = edge_index[1].astype(jnp.int32)

    part, cap = _make_partition(e)
    csrc, cloc, nblk, deg_flat = part(src, dst)
    deg16 = deg_flat.reshape(NPAD, LANES)[:n]

    agg = _make_agg(n, cap, x.shape[1], 128)

    g0 = _scale0(x, deg16)
    a1 = agg(g0, csrc, cloc, nblk)[:n]
    h2, g2 = _fused_mm(a1, x, deg16, b1.reshape(1, -1), W1, W2)
    a2 = agg(g2, csrc, cloc, nblk)[:n]
    return _combine_final(a2, h2, deg16, b2.reshape(1, -1))
